# Initial kernel scaffold; baseline (speedup 1.0000x reference)
import jax
import jax.numpy as jnp
from jax.experimental import pallas as pl


def kernel(x, codebook):
    raise NotImplementedError("write your pallas kernel here")


# speedup vs baseline: 1.0808x; 1.0808x over previous
import jax
import jax.numpy as jnp
from jax.experimental import pallas as pl
from jax.experimental.pallas import tpu as pltpu

_B, _C, _HW = 16, 64, 576
_K = 1024
_NTOT = _B * _HW


def _vq_body(x_ref, cb_ref, enc_ref, q_ref, loss_ref, perp_ref,
             counts_ref, acc_ref):
    b = pl.program_id(0)
    xb = x_ref[0]
    cb = cb_ref[...]
    prod = jax.lax.dot_general(cb, xb, (((0,), (0,)), ((), ())),
                               preferred_element_type=jnp.float32)
    csq = jnp.sum(cb * cb, axis=0, keepdims=True)
    csq_col = csq.T
    xsq = jnp.sum(xb * xb, axis=0, keepdims=True)
    d = (xsq - 2.0 * prod) + csq_col

    kio = jax.lax.broadcasted_iota(jnp.int32, (_K, _HW), 0)
    m = jnp.min(d, axis=0, keepdims=True)
    idx_row = jnp.min(jnp.where(d == m, kio, _K), axis=0, keepdims=True)
    oh_t = (kio == idx_row).astype(jnp.float32)
    q = jnp.dot(cb, oh_t, preferred_element_type=jnp.float32)
    q_ref[0] = q

    idx_col = idx_row.T
    lio = jax.lax.broadcasted_iota(jnp.int32, (_HW, _K), 1)
    enc_ref[...] = (lio == idx_col).astype(jnp.float32)

    diff = q - xb
    part = jnp.sum(diff * diff)
    cnt = jnp.sum(oh_t, axis=1, keepdims=True)

    @pl.when(b == 0)
    def _():
        acc_ref[0] = part
        counts_ref[...] = cnt

    @pl.when(b != 0)
    def _():
        acc_ref[0] = acc_ref[0] + part
        counts_ref[...] = counts_ref[...] + cnt

    @pl.when(b == _B - 1)
    def _():
        loss_ref[0] = 1.25 * acc_ref[0] / float(_NTOT * _C)
        p = counts_ref[...] / float(_NTOT)
        ent = jnp.sum(p * jnp.log(p + 1e-10))
        perp_ref[0] = jnp.exp(-ent)


@jax.jit
def kernel(x, codebook):
    xr = x.reshape(_B, _C, _HW)
    enc, q, loss, perp = pl.pallas_call(
        _vq_body,
        grid=(_B,),
        in_specs=[
            pl.BlockSpec((1, _C, _HW), lambda b: (b, 0, 0)),
            pl.BlockSpec((_C, _K), lambda b: (0, 0)),
        ],
        out_specs=[
            pl.BlockSpec((_HW, _K), lambda b: (b, 0)),
            pl.BlockSpec((1, _C, _HW), lambda b: (b, 0, 0)),
            pl.BlockSpec(memory_space=pltpu.SMEM),
            pl.BlockSpec(memory_space=pltpu.SMEM),
        ],
        out_shape=[
            jax.ShapeDtypeStruct((_NTOT, _K), jnp.float32),
            jax.ShapeDtypeStruct((_B, _C, _HW), jnp.float32),
            jax.ShapeDtypeStruct((1,), jnp.float32),
            jax.ShapeDtypeStruct((1,), jnp.float32),
        ],
        scratch_shapes=[
            pltpu.VMEM((_K, 1), jnp.float32),
            pltpu.SMEM((1,), jnp.float32),
        ],
        compiler_params=pltpu.CompilerParams(
            dimension_semantics=("arbitrary",)),
    )(xr, codebook)
    return (loss[0], q.reshape(16, 64, 24, 24), perp[0], enc)


# speedup vs baseline: 1.2025x; 1.1126x over previous
import jax
import jax.numpy as jnp
from jax.experimental import pallas as pl
from jax.experimental.pallas import tpu as pltpu

_B, _C, _HW = 16, 64, 576
_K = 1024
_NTOT = _B * _HW


def _vq_body(x_ref, cb_ref, enc_ref, q_ref, loss_ref, perp_ref,
             counts_ref, acc_ref):
    b = pl.program_id(0)
    xb = x_ref[0]
    cb = cb_ref[...]
    xbT = xb.T
    p2 = jnp.dot(xbT, -2.0 * cb, preferred_element_type=jnp.float32)
    xsq = jnp.sum(xbT * xbT, axis=1, keepdims=True)
    csq = jnp.sum(cb * cb, axis=0, keepdims=True)
    d = (xsq + p2) + csq

    m = jnp.min(d, axis=1, keepdims=True)
    lio = jax.lax.broadcasted_iota(jnp.int32, (_HW, _K), 1)
    idx = jnp.min(jnp.where(d == m, lio, _K), axis=1, keepdims=True)
    enc = (lio == idx).astype(jnp.float32)
    enc_ref[...] = enc

    qn = jnp.dot(enc, cb.T, preferred_element_type=jnp.float32)
    q_ref[0] = qn.T

    diffn = qn - xbT
    part = jnp.sum(diffn * diffn)
    cnt = jnp.sum(enc, axis=0, keepdims=True)

    @pl.when(b == 0)
    def _():
        acc_ref[0] = part
        counts_ref[...] = cnt

    @pl.when(b != 0)
    def _():
        acc_ref[0] = acc_ref[0] + part
        counts_ref[...] = counts_ref[...] + cnt

    @pl.when(b == _B - 1)
    def _():
        loss_ref[0] = 1.25 * acc_ref[0] / float(_NTOT * _C)
        p = counts_ref[...] / float(_NTOT)
        ent = jnp.sum(p * jnp.log(p + 1e-10))
        perp_ref[0] = jnp.exp(-ent)


@jax.jit
def kernel(x, codebook):
    xr = x.reshape(_B, _C, _HW)
    enc, q, loss, perp = pl.pallas_call(
        _vq_body,
        grid=(_B,),
        in_specs=[
            pl.BlockSpec((1, _C, _HW), lambda b: (b, 0, 0)),
            pl.BlockSpec((_C, _K), lambda b: (0, 0)),
        ],
        out_specs=[
            pl.BlockSpec((_HW, _K), lambda b: (b, 0)),
            pl.BlockSpec((1, _C, _HW), lambda b: (b, 0, 0)),
            pl.BlockSpec(memory_space=pltpu.SMEM),
            pl.BlockSpec(memory_space=pltpu.SMEM),
        ],
        out_shape=[
            jax.ShapeDtypeStruct((_NTOT, _K), jnp.float32),
            jax.ShapeDtypeStruct((_B, _C, _HW), jnp.float32),
            jax.ShapeDtypeStruct((1,), jnp.float32),
            jax.ShapeDtypeStruct((1,), jnp.float32),
        ],
        scratch_shapes=[
            pltpu.VMEM((1, _K), jnp.float32),
            pltpu.SMEM((1,), jnp.float32),
        ],
        compiler_params=pltpu.CompilerParams(
            dimension_semantics=("arbitrary",)),
    )(xr, codebook)
    return (loss[0], q.reshape(16, 64, 24, 24), perp[0], enc)


# speedup vs baseline: 1.4368x; 1.1948x over previous
import jax
import jax.numpy as jnp
from jax.experimental import pallas as pl
from jax.experimental.pallas import tpu as pltpu

_B, _C, _HW = 16, 64, 576
_K = 1024
_NTOT = _B * _HW
_PB = 2
_N2 = _PB * _HW
_G = _B // _PB


def _vq_body(x_ref, cb_ref, enc_ref, q_ref, loss_ref, perp_ref,
             counts_ref, acc_ref):
    s = pl.program_id(0)
    cb = cb_ref[...]
    xT = jnp.concatenate([x_ref[0].T, x_ref[1].T], axis=0)
    p2 = jnp.dot(xT, -2.0 * cb, preferred_element_type=jnp.float32)
    xsq = jnp.sum(xT * xT, axis=1, keepdims=True)
    csq = jnp.sum(cb * cb, axis=0, keepdims=True)
    d = (xsq + p2) + csq

    m = jnp.min(d, axis=1, keepdims=True)
    maskb = d == m
    mask = maskb.astype(jnp.float32)
    enc_ref[...] = mask

    rhs = jnp.concatenate(
        [cb.T, jnp.full((_K, 1), 1.0, jnp.float32)], axis=1)
    qx = jnp.dot(mask, rhs, preferred_element_type=jnp.float32)
    qn = qx[:, :_C]
    q_ref[0] = qn[:_HW].T
    q_ref[1] = qn[_HW:].T

    diff = qn - xT
    part = jnp.sum(diff * diff)
    cnt = jnp.sum(mask, axis=0, keepdims=True)

    @pl.when(s == 0)
    def _():
        acc_ref[0] = part
        counts_ref[...] = cnt

    @pl.when(s != 0)
    def _():
        acc_ref[0] = acc_ref[0] + part
        counts_ref[...] = counts_ref[...] + cnt

    tie = jnp.max(qx[:, _C:]) > 1.5

    @pl.when(tie)
    def _():
        lio = jax.lax.broadcasted_iota(jnp.int32, (_N2, _K), 1)
        idx = jnp.min(jnp.where(maskb, lio, _K), axis=1, keepdims=True)
        encf = (lio == idx).astype(jnp.float32)
        enc_ref[...] = encf
        q2 = jnp.dot(encf, cb.T, preferred_element_type=jnp.float32)
        q_ref[0] = q2[:_HW].T
        q_ref[1] = q2[_HW:].T
        d2 = q2 - xT
        part2 = jnp.sum(d2 * d2)
        cnt2 = jnp.sum(encf, axis=0, keepdims=True)
        acc_ref[0] = acc_ref[0] + (part2 - part)
        counts_ref[...] = counts_ref[...] + (cnt2 - cnt)

    @pl.when(s == _G - 1)
    def _():
        loss_ref[0] = 1.25 * acc_ref[0] / float(_NTOT * _C)
        p = counts_ref[...] / float(_NTOT)
        ent = jnp.sum(p * jnp.log(p + 1e-10))
        perp_ref[0] = jnp.exp(-ent)


@jax.jit
def kernel(x, codebook):
    xr = x.reshape(_B, _C, _HW)
    enc, q, loss, perp = pl.pallas_call(
        _vq_body,
        grid=(_G,),
        in_specs=[
            pl.BlockSpec((_PB, _C, _HW), lambda s: (s, 0, 0)),
            pl.BlockSpec((_C, _K), lambda s: (0, 0)),
        ],
        out_specs=[
            pl.BlockSpec((_N2, _K), lambda s: (s, 0)),
            pl.BlockSpec((_PB, _C, _HW), lambda s: (s, 0, 0)),
            pl.BlockSpec(memory_space=pltpu.SMEM),
            pl.BlockSpec(memory_space=pltpu.SMEM),
        ],
        out_shape=[
            jax.ShapeDtypeStruct((_NTOT, _K), jnp.float32),
            jax.ShapeDtypeStruct((_B, _C, _HW), jnp.float32),
            jax.ShapeDtypeStruct((1,), jnp.float32),
            jax.ShapeDtypeStruct((1,), jnp.float32),
        ],
        scratch_shapes=[
            pltpu.VMEM((1, _K), jnp.float32),
            pltpu.SMEM((1,), jnp.float32),
        ],
        compiler_params=pltpu.CompilerParams(
            dimension_semantics=("arbitrary",)),
    )(xr, codebook)
    return (loss[0], q.reshape(16, 64, 24, 24), perp[0], enc)


# speedup vs baseline: 1.4742x; 1.0260x over previous
import jax
import jax.numpy as jnp
from jax.experimental import pallas as pl
from jax.experimental.pallas import tpu as pltpu

_B, _C, _HW = 16, 64, 576
_K = 1024
_NTOT = _B * _HW
_PB = 4
_N2 = _PB * _HW
_G = _B // _PB


def _vq_body(x_ref, cb_ref, enc_ref, q_ref, loss_ref, perp_ref,
             counts_ref, acc_ref):
    s = pl.program_id(0)
    cb = cb_ref[...]
    xT = jnp.concatenate([x_ref[i].T for i in range(_PB)], axis=0)
    p2 = jnp.dot(xT, -2.0 * cb, preferred_element_type=jnp.float32)
    xsq = jnp.sum(xT * xT, axis=1, keepdims=True)
    csq = jnp.sum(cb * cb, axis=0, keepdims=True)
    d = (xsq + p2) + csq

    m = jnp.min(d, axis=1, keepdims=True)
    maskb = d == m
    mask = maskb.astype(jnp.float32)
    enc_ref[...] = mask

    rhs = jnp.concatenate(
        [cb.T, jnp.full((_K, 1), 1.0, jnp.float32)], axis=1)
    qx = jnp.dot(mask, rhs, preferred_element_type=jnp.float32)
    qn = qx[:, :_C]
    for i in range(_PB):
        q_ref[i] = qn[i * _HW:(i + 1) * _HW].T

    diff = qn - xT
    part = jnp.sum(diff * diff)
    cnt = jnp.sum(mask, axis=0, keepdims=True)

    @pl.when(s == 0)
    def _():
        acc_ref[0] = part
        counts_ref[...] = cnt

    @pl.when(s != 0)
    def _():
        acc_ref[0] = acc_ref[0] + part
        counts_ref[...] = counts_ref[...] + cnt

    tie = jnp.max(qx[:, _C:]) > 1.5

    @pl.when(tie)
    def _():
        lio = jax.lax.broadcasted_iota(jnp.int32, (_N2, _K), 1)
        idx = jnp.min(jnp.where(maskb, lio, _K), axis=1, keepdims=True)
        encf = (lio == idx).astype(jnp.float32)
        enc_ref[...] = encf
        q2 = jnp.dot(encf, cb.T, preferred_element_type=jnp.float32)
        for i in range(_PB):
            q_ref[i] = q2[i * _HW:(i + 1) * _HW].T
        d2 = q2 - xT
        part2 = jnp.sum(d2 * d2)
        cnt2 = jnp.sum(encf, axis=0, keepdims=True)
        acc_ref[0] = acc_ref[0] + (part2 - part)
        counts_ref[...] = counts_ref[...] + (cnt2 - cnt)

    @pl.when(s == _G - 1)
    def _():
        loss_ref[0] = 1.25 * acc_ref[0] / float(_NTOT * _C)
        p = counts_ref[...] / float(_NTOT)
        ent = jnp.sum(p * jnp.log(p + 1e-10))
        perp_ref[0] = jnp.exp(-ent)


@jax.jit
def kernel(x, codebook):
    xr = x.reshape(_B, _C, _HW)
    enc, q, loss, perp = pl.pallas_call(
        _vq_body,
        grid=(_G,),
        in_specs=[
            pl.BlockSpec((_PB, _C, _HW), lambda s: (s, 0, 0)),
            pl.BlockSpec((_C, _K), lambda s: (0, 0)),
        ],
        out_specs=[
            pl.BlockSpec((_N2, _K), lambda s: (s, 0)),
            pl.BlockSpec((_PB, _C, _HW), lambda s: (s, 0, 0)),
            pl.BlockSpec(memory_space=pltpu.SMEM),
            pl.BlockSpec(memory_space=pltpu.SMEM),
        ],
        out_shape=[
            jax.ShapeDtypeStruct((_NTOT, _K), jnp.float32),
            jax.ShapeDtypeStruct((_B, _C, _HW), jnp.float32),
            jax.ShapeDtypeStruct((1,), jnp.float32),
            jax.ShapeDtypeStruct((1,), jnp.float32),
        ],
        scratch_shapes=[
            pltpu.VMEM((1, _K), jnp.float32),
            pltpu.SMEM((1,), jnp.float32),
        ],
        compiler_params=pltpu.CompilerParams(
            dimension_semantics=("arbitrary",)),
    )(xr, codebook)
    return (loss[0], q.reshape(16, 64, 24, 24), perp[0], enc)


# speedup vs baseline: 1.5652x; 1.0617x over previous
import jax
import jax.numpy as jnp
from jax.experimental import pallas as pl
from jax.experimental.pallas import tpu as pltpu

_B, _C, _HW = 16, 64, 576
_K = 1024
_NTOT = _B * _HW
_PB = 4
_N2 = _PB * _HW
_G = _B // _PB


def _vq_body(x_ref, cb_ref, enc_ref, q_ref, loss_ref, perp_ref,
             counts_ref, acc_ref):
    s = pl.program_id(0)
    cb = cb_ref[...]
    xT = jnp.concatenate([x_ref[i].T for i in range(_PB)], axis=0)
    p2 = jnp.dot(xT, -2.0 * cb, preferred_element_type=jnp.float32)
    xsq = jnp.sum(xT * xT, axis=1, keepdims=True)
    csq = jnp.sum(cb * cb, axis=0, keepdims=True)
    d = (xsq + p2) + csq

    m = jnp.min(d, axis=1, keepdims=True)
    maskb = d == m
    mask = maskb.astype(jnp.float32)
    enc_ref[...] = mask

    qn = jnp.dot(mask, cb.T, preferred_element_type=jnp.float32)
    for i in range(_PB):
        q_ref[i] = qn[i * _HW:(i + 1) * _HW].T

    diff = qn - xT
    part = jnp.sum(diff * diff)
    cnt = jnp.sum(mask, axis=0, keepdims=True)

    @pl.when(s == 0)
    def _():
        acc_ref[0] = part
        counts_ref[...] = cnt

    @pl.when(s != 0)
    def _():
        acc_ref[0] = acc_ref[0] + part
        counts_ref[...] = counts_ref[...] + cnt

    tie = jnp.sum(cnt) > float(_N2) + 0.5

    @pl.when(tie)
    def _():
        lio = jax.lax.broadcasted_iota(jnp.int32, (_N2, _K), 1)
        idx = jnp.min(jnp.where(maskb, lio, _K), axis=1, keepdims=True)
        encf = (lio == idx).astype(jnp.float32)
        enc_ref[...] = encf
        q2 = jnp.dot(encf, cb.T, preferred_element_type=jnp.float32)
        for i in range(_PB):
            q_ref[i] = q2[i * _HW:(i + 1) * _HW].T
        d2 = q2 - xT
        part2 = jnp.sum(d2 * d2)
        cnt2 = jnp.sum(encf, axis=0, keepdims=True)
        acc_ref[0] = acc_ref[0] + (part2 - part)
        counts_ref[...] = counts_ref[...] + (cnt2 - cnt)

    @pl.when(s == _G - 1)
    def _():
        loss_ref[0] = 1.25 * acc_ref[0] / float(_NTOT * _C)
        p = counts_ref[...] / float(_NTOT)
        ent = jnp.sum(p * jnp.log(p + 1e-10))
        perp_ref[0] = jnp.exp(-ent)


@jax.jit
def kernel(x, codebook):
    xr = x.reshape(_B, _C, _HW)
    enc, q, loss, perp = pl.pallas_call(
        _vq_body,
        grid=(_G,),
        in_specs=[
            pl.BlockSpec((_PB, _C, _HW), lambda s: (s, 0, 0)),
            pl.BlockSpec((_C, _K), lambda s: (0, 0)),
        ],
        out_specs=[
            pl.BlockSpec((_N2, _K), lambda s: (s, 0)),
            pl.BlockSpec((_PB, _C, _HW), lambda s: (s, 0, 0)),
            pl.BlockSpec(memory_space=pltpu.SMEM),
            pl.BlockSpec(memory_space=pltpu.SMEM),
        ],
        out_shape=[
            jax.ShapeDtypeStruct((_NTOT, _K), jnp.float32),
            jax.ShapeDtypeStruct((_B, _C, _HW), jnp.float32),
            jax.ShapeDtypeStruct((1,), jnp.float32),
            jax.ShapeDtypeStruct((1,), jnp.float32),
        ],
        scratch_shapes=[
            pltpu.VMEM((1, _K), jnp.float32),
            pltpu.SMEM((1,), jnp.float32),
        ],
        compiler_params=pltpu.CompilerParams(
            dimension_semantics=("arbitrary",)),
    )(xr, codebook)
    return (loss[0], q.reshape(16, 64, 24, 24), perp[0], enc)
